# baseline (speedup 1.0000x reference)
import functools

import jax
import jax.numpy as jnp
from jax import lax
from jax.experimental import pallas as pl
from jax.experimental.pallas import tpu as pltpu
from jax.experimental.pallas import tpu_sc as plsc

_B = 4096
_S = 50
_D = 64
_NC = 2
_NS = 16
_NW = _NC * _NS
_SEQ_W = _B // _NW
_ROWS_W = _SEQ_W * _S
_CROWS = 128
_NCHUNK = _ROWS_W // _CROWS


def _sc_body(tokens_hbm, labels_hbm, table_hbm, ps_hbm,
             hyb_out, tok_out,
             idx_v, cidx_v, lbl_v, rows_v, hyb_v, sem_tok, sem_ps):
    c = lax.axis_index("c")
    s = lax.axis_index("s")
    wid = s * _NC + c
    seq_base = wid * _SEQ_W
    w_row_base = wid * _ROWS_W

    pltpu.sync_copy(labels_hbm.at[pl.ds(seq_base, _SEQ_W)], lbl_v)

    @pl.loop(0, _NCHUNK)
    def _chunk(ci):
        local_base = ci * _CROWS
        row_base = pl.multiple_of(w_row_base + local_base, _CROWS)

        pltpu.sync_copy(tokens_hbm.at[pl.ds(row_base, _CROWS)], idx_v)
        pltpu.async_copy(table_hbm.at[idx_v], rows_v, sem_tok).wait()
        pltpu.sync_copy(rows_v, tok_out.at[pl.ds(row_base, _CROWS)])

        @pl.loop(0, _CROWS // 16)
        def _mkidx(v):
            flat = jnp.full((16,), local_base + v * 16, jnp.int32) + lax.iota(
                jnp.int32, 16)
            pos = flat % _S
            seq_local = lax.div(flat, jnp.int32(_S))
            lbl = plsc.load_gather(lbl_v, [seq_local])
            cidx_v[pl.ds(v * 16, 16)] = lbl * _S + pos

        pltpu.async_copy(ps_hbm.at[cidx_v], hyb_v, sem_ps).wait()

        @pl.loop(0, _CROWS)
        def _add(r):
            for k in range(_D // 16):
                sl = pl.ds(k * 16, 16)
                hyb_v[r, sl] = hyb_v[r, sl] + rows_v[r, sl]

        pltpu.sync_copy(hyb_v, hyb_out.at[pl.ds(row_base, _CROWS)])


@jax.jit
def _sc_call(tokens_flat, labels, token_table, ps):
    mesh = plsc.VectorSubcoreMesh(core_axis_name="c", subcore_axis_name="s")
    run = pl.kernel(
        _sc_body,
        out_type=(
            jax.ShapeDtypeStruct((_B * _S, _D), jnp.float32),
            jax.ShapeDtypeStruct((_B * _S, _D), jnp.float32),
        ),
        mesh=mesh,
        scratch_types=[
            pltpu.VMEM((_CROWS,), jnp.int32),
            pltpu.VMEM((_CROWS,), jnp.int32),
            pltpu.VMEM((_SEQ_W,), jnp.int32),
            pltpu.VMEM((_CROWS, _D), jnp.float32),
            pltpu.VMEM((_CROWS, _D), jnp.float32),
            pltpu.SemaphoreType.DMA,
            pltpu.SemaphoreType.DMA,
        ],
        compiler_params=pltpu.CompilerParams(
            use_tc_tiling_on_sc=False, needs_layout_passes=False),
    )
    return run(tokens_flat, labels, token_table, ps)


def kernel(tokens, labels, token_table, style_table, posit_table):
    seq = tokens.shape[1]
    ps = (style_table[:, None, :] + posit_table[:seq][None, :, :]).reshape(
        -1, _D)
    tokens_flat = tokens.reshape(-1).astype(jnp.int32)
    hyb, tok = _sc_call(tokens_flat, labels.astype(jnp.int32), token_table,
                        ps)
    return hyb.reshape(_B, seq, _D), tok.reshape(_B, seq, _D)


# speedup vs baseline: 1.0107x; 1.0107x over previous
import functools

import jax
import jax.numpy as jnp
from jax import lax
from jax.experimental import pallas as pl
from jax.experimental.pallas import tpu as pltpu
from jax.experimental.pallas import tpu_sc as plsc

_B = 4096
_S = 50
_D = 64
_NC = 2
_NS = 16
_NW = _NC * _NS
_SEQ_W = _B // _NW
_ROWS_W = _SEQ_W * _S
_CROWS = 128
_NCHUNK = _ROWS_W // _CROWS


def _sc_body(tokens_hbm, labels_hbm, table_hbm, ps_hbm,
             hyb_out, tok_out,
             idx0, idx1, cidx_v, lbl_v, rows0, rows1, hyb0, hyb1,
             s_idx0, s_idx1, s_tok0, s_tok1, s_ps0, s_ps1,
             s_to0, s_to1, s_ho0, s_ho1):
    c = lax.axis_index("c")
    s = lax.axis_index("s")
    wid = s * _NC + c
    seq_base = wid * _SEQ_W
    w_row_base = wid * _ROWS_W

    idx_v = (idx0, idx1)
    rows_v = (rows0, rows1)
    hyb_v = (hyb0, hyb1)
    s_idx = (s_idx0, s_idx1)
    s_tok = (s_tok0, s_tok1)
    s_ps = (s_ps0, s_ps1)
    s_to = (s_to0, s_to1)
    s_ho = (s_ho0, s_ho1)

    pltpu.sync_copy(labels_hbm.at[pl.ds(seq_base, _SEQ_W)], lbl_v)

    @pl.loop(0, _ROWS_W // 16)
    def _mkidx(v):
        flat = jnp.full((16,), v * 16, jnp.int32) + lax.iota(jnp.int32, 16)
        pos = flat % _S
        seq_local = lax.div(flat, jnp.int32(_S))
        lbl = plsc.load_gather(lbl_v, [seq_local])
        g = lax.div(v, jnp.int32(_CROWS // 16))
        j = lax.rem(v, jnp.int32(_CROWS // 16))
        cidx_v[g, pl.ds(j * 16, 16)] = lbl * _S + pos

    def row_slice(g):
        return pl.ds(pl.multiple_of(w_row_base + g * _CROWS, _CROWS), _CROWS)

    pltpu.async_copy(tokens_hbm.at[row_slice(0)], idx_v[0], s_idx[0])

    def stage(g, b):
        nb = 1 - b
        @pl.when(g >= 2)
        def _drain():
            pltpu.make_async_copy(rows_v[b], tok_out.at[row_slice(g)],
                                  s_to[b]).wait()
            pltpu.make_async_copy(hyb_v[b], hyb_out.at[row_slice(g)],
                                  s_ho[b]).wait()
        pltpu.async_copy(ps_hbm.at[cidx_v.at[g]], hyb_v[b], s_ps[b])
        pltpu.make_async_copy(tokens_hbm.at[row_slice(g)], idx_v[b],
                              s_idx[b]).wait()
        pltpu.async_copy(table_hbm.at[idx_v[b]], rows_v[b], s_tok[b])
        @pl.when(g + 1 < _NCHUNK)
        def _pref():
            pltpu.async_copy(tokens_hbm.at[row_slice(g + 1)], idx_v[nb],
                             s_idx[nb])
        pltpu.make_async_copy(table_hbm.at[idx_v[b]], rows_v[b],
                              s_tok[b]).wait()
        pltpu.async_copy(rows_v[b], tok_out.at[row_slice(g)], s_to[b])
        pltpu.make_async_copy(ps_hbm.at[cidx_v.at[g]], hyb_v[b],
                              s_ps[b]).wait()

        @pl.loop(0, _CROWS)
        def _add(r):
            for k in range(_D // 16):
                sl = pl.ds(k * 16, 16)
                hyb_v[b][r, sl] = hyb_v[b][r, sl] + rows_v[b][r, sl]

        pltpu.async_copy(hyb_v[b], hyb_out.at[row_slice(g)], s_ho[b])

    @pl.loop(0, _NCHUNK, step=2)
    def _chunk(g):
        stage(g, 0)
        stage(g + 1, 1)

    for b in range(2):
        pltpu.make_async_copy(rows_v[b], tok_out.at[row_slice(0)],
                              s_to[b]).wait()
        pltpu.make_async_copy(hyb_v[b], hyb_out.at[row_slice(0)],
                              s_ho[b]).wait()


@jax.jit
def _sc_call(tokens_flat, labels, token_table, ps):
    mesh = plsc.VectorSubcoreMesh(core_axis_name="c", subcore_axis_name="s")
    run = pl.kernel(
        _sc_body,
        out_type=(
            jax.ShapeDtypeStruct((_B * _S, _D), jnp.float32),
            jax.ShapeDtypeStruct((_B * _S, _D), jnp.float32),
        ),
        mesh=mesh,
        scratch_types=[
            pltpu.VMEM((_CROWS,), jnp.int32),
            pltpu.VMEM((_CROWS,), jnp.int32),
            pltpu.VMEM((_NCHUNK, _CROWS), jnp.int32),
            pltpu.VMEM((_SEQ_W,), jnp.int32),
            pltpu.VMEM((_CROWS, _D), jnp.float32),
            pltpu.VMEM((_CROWS, _D), jnp.float32),
            pltpu.VMEM((_CROWS, _D), jnp.float32),
            pltpu.VMEM((_CROWS, _D), jnp.float32),
        ] + [pltpu.SemaphoreType.DMA] * 10,
        compiler_params=pltpu.CompilerParams(
            use_tc_tiling_on_sc=False, needs_layout_passes=False),
    )
    return run(tokens_flat, labels, token_table, ps)


def kernel(tokens, labels, token_table, style_table, posit_table):
    seq = tokens.shape[1]
    ps = (style_table[:, None, :] + posit_table[:seq][None, :, :]).reshape(
        -1, _D)
    tokens_flat = tokens.reshape(-1).astype(jnp.int32)
    hyb, tok = _sc_call(tokens_flat, labels.astype(jnp.int32), token_table,
                        ps)
    return hyb.reshape(_B, seq, _D), tok.reshape(_B, seq, _D)
